# Initial kernel scaffold; baseline (speedup 1.0000x reference)
import jax
import jax.numpy as jnp
from jax.experimental import pallas as pl


def kernel(x_list1, x_list2, adj1, adj2, W_init, b_init, g_init, be_init, W_mid, b_mid, g_mid, be_mid, W_out, b_out, fc1w1, fc1b1, fc1w2, fc1b2, awsw1, awsb1, awsw2, awsb2, fcw1, fcb1, fcw2, fcb2):
    raise NotImplementedError("write your pallas kernel here")


# speedup vs baseline: 9.7460x; 9.7460x over previous
import functools

import jax
import jax.numpy as jnp
from jax import lax
from jax.experimental import pallas as pl
from jax.experimental.pallas import tpu as pltpu
from jax.experimental.pallas import tpu_sc as plsc

_N = 10000
_NP = 10240
_E = 320000
_NC = 2
_NS = 16
_CPT = 160
_EP = _CPT * _NS * 128
_BLK = 1024
_GRID = _NP // _BLK
_BNS = float(1.0 / (1.0 + 1e-5) ** 0.5)
_KG = 16
_LAST = 7
_LBI = (_N - 1) // 8


@functools.lru_cache(None)
def _make_agg(F):
    mesh = plsc.VectorSubcoreMesh(
        core_axis_name="c", subcore_axis_name="s",
        num_cores=_NC, num_subcores=_NS)
    rpt = _NP // _NS

    @functools.partial(
        pl.kernel,
        out_type=jax.ShapeDtypeStruct((_NC, _NP, F), jnp.float32),
        mesh=mesh,
        scratch_types=[
            pltpu.VMEM_SHARED((_NP, F), jnp.float32),
            pltpu.VMEM((_KG, 128), jnp.int32),
            pltpu.VMEM((_KG, 128), jnp.int32),
            pltpu.VMEM((128, F), jnp.float32),
            pltpu.VMEM((16, F), jnp.float32),
            pltpu.SemaphoreType.DMA,
        ],
    )
    def agg(hs_hbm, sidx_hbm, didx_hbm, out_hbm,
            acc, sidx_v, didx_v, rows_v, zero_v, gsem):
        cid = lax.axis_index("c")
        sid = lax.axis_index("s")
        zv = jnp.zeros((16,), jnp.float32)
        for r in range(16):
            for c in range(F // 16):
                zero_v[r, pl.ds(c * 16, 16)] = zv

        def zero_body(t, carry):
            pltpu.sync_copy(zero_v, acc.at[pl.ds(sid * rpt + t * 16, 16)])
            return carry
        lax.fori_loop(0, rpt // 16, zero_body, 0)

        row0 = (cid * _NS + sid) * _CPT
        plsc.subcore_barrier()

        def group_body(gq, carry):
            g0 = row0 + gq * _KG
            pltpu.sync_copy(sidx_hbm.at[pl.ds(g0, _KG)], sidx_v)
            pltpu.sync_copy(didx_hbm.at[pl.ds(g0, _KG)], didx_v)

            def body(j, c2):
                pltpu.async_copy(hs_hbm.at[sidx_v.at[j]], rows_v, gsem).wait()
                pltpu.sync_copy(rows_v, acc.at[didx_v.at[j]], add=True)
                return c2
            lax.fori_loop(0, _KG, body, 0)
            return carry
        lax.fori_loop(0, _CPT // _KG, group_body, 0)
        plsc.subcore_barrier()

        pltpu.sync_copy(acc.at[pl.ds(sid * rpt, rpt)],
                        out_hbm.at[cid, pl.ds(sid * rpt, rpt)])

    return agg


@functools.lru_cache(None)
def _make_deg():
    F = 128
    mesh = plsc.VectorSubcoreMesh(
        core_axis_name="c", subcore_axis_name="s",
        num_cores=_NC, num_subcores=_NS)
    rpt = _NP // _NS

    @functools.partial(
        pl.kernel,
        out_type=jax.ShapeDtypeStruct((_NC, _NP, F), jnp.float32),
        mesh=mesh,
        scratch_types=[
            pltpu.VMEM_SHARED((_NP, F), jnp.float32),
            pltpu.VMEM((_KG, 128), jnp.int32),
            pltpu.VMEM((128, F), jnp.float32),
            pltpu.VMEM((16, F), jnp.float32),
        ],
    )
    def deg(didx_hbm, out_hbm, acc, didx_v, ones_v, zero_v):
        cid = lax.axis_index("c")
        sid = lax.axis_index("s")
        zv = jnp.zeros((16,), jnp.float32)
        ov = jnp.ones((16,), jnp.float32)
        for r in range(16):
            for c in range(F // 16):
                zero_v[r, pl.ds(c * 16, 16)] = zv

        def ones_body(r, carry):
            for c in range(F // 16):
                ones_v[r, pl.ds(c * 16, 16)] = ov
            return carry
        lax.fori_loop(0, 128, ones_body, 0)

        def zero_body(t, carry):
            pltpu.sync_copy(zero_v, acc.at[pl.ds(sid * rpt + t * 16, 16)])
            return carry
        lax.fori_loop(0, rpt // 16, zero_body, 0)

        row0 = (cid * _NS + sid) * _CPT
        plsc.subcore_barrier()

        def group_body(gq, carry):
            g0 = row0 + gq * _KG
            pltpu.sync_copy(didx_hbm.at[pl.ds(g0, _KG)], didx_v)

            def body(j, c2):
                pltpu.sync_copy(ones_v, acc.at[didx_v.at[j]], add=True)
                return c2
            lax.fori_loop(0, _KG, body, 0)
            return carry
        lax.fori_loop(0, _CPT // _KG, group_body, 0)
        plsc.subcore_barrier()

        pltpu.sync_copy(acc.at[pl.ds(sid * rpt, rpt)],
                        out_hbm.at[cid, pl.ds(sid * rpt, rpt)])

    return deg


def _blk3(w):
    return pl.BlockSpec((2, _BLK, w), lambda i: (0, i, 0))


def _last3(w):
    lbi = (_N - 1) // 8
    return pl.BlockSpec((2, 8, w), lambda i, _l=lbi: (0, _l, 0))


def _full(shape):
    nd = len(shape)
    return pl.BlockSpec(shape, lambda i, _n=nd: (0,) * _n)


def _fusion_w(a_row, b_row, wa, ba, wb, bb):
    la = jax.nn.sigmoid(jnp.dot(a_row, wa) + ba)
    lb = jax.nn.sigmoid(jnp.dot(b_row, wb) + bb)
    s = jnp.abs(la) + jnp.abs(lb)
    return la / s, lb / s


def _tc_b(x1, x2, W, cnt):
    def body(x1_ref, x2_ref, w_ref, cnt_ref, out_ref):
        dinv = lax.rsqrt(1.0 + cnt_ref[:, :, 0:1])
        w0 = w_ref[0]
        w1 = w_ref[1]
        h1 = jnp.concatenate(
            [jnp.dot(x1_ref[0], w0), jnp.dot(x1_ref[1], w1)], axis=-1)
        h2 = jnp.concatenate(
            [jnp.dot(x2_ref[0], w0), jnp.dot(x2_ref[1], w1)], axis=-1)
        out_ref[0] = dinv[0] * h1
        out_ref[1] = dinv[1] * h2

    return pl.pallas_call(
        body,
        out_shape=jax.ShapeDtypeStruct((_NC, _NP, 128), jnp.float32),
        grid=(_GRID,),
        in_specs=[_blk3(128), _blk3(128), _full((2, 128, 64)), _blk3(128)],
        out_specs=_blk3(128),
    )(x1, x2, W, cnt)


def _tc_d(S, hs, cnt, bcat, gcat, becat, fw1, fb1, fw2, fb2):
    def body(S_ref, hs_ref, cnt_ref, Sl_ref, hsl_ref, cntl_ref,
             b_ref, g_ref, be_ref, fw1_ref, fb1_ref, fw2_ref, fb2_ref,
             out_ref):
        b = b_ref[...]
        g = g_ref[...]
        be = be_ref[...]

        def xcalc(Sv, hv, cv):
            dinv = lax.rsqrt(1.0 + cv[:, :, 0:1])
            m = dinv * (Sv + hv) + b
            return jax.nn.relu((m * _BNS) * g + be), dinv

        x_blk, dinv = xcalc(S_ref[...], hs_ref[...], cnt_ref[...])
        x_last, _ = xcalc(Sl_ref[...], hsl_ref[...], cntl_ref[...])
        w1, w2 = _fusion_w(x_last[0, _LAST:_LAST + 1], x_last[1, _LAST:_LAST + 1],
                           fw1_ref[...], fb1_ref[...], fw2_ref[...], fb2_ref[...])
        xa = w1 * x_blk[0] + w2 * x_blk[1]
        out_ref[0] = dinv[0] * xa
        out_ref[1] = dinv[1] * xa

    return pl.pallas_call(
        body,
        out_shape=jax.ShapeDtypeStruct((_NC, _NP, 128), jnp.float32),
        grid=(_GRID,),
        in_specs=[_blk3(128), _blk3(128), _blk3(128),
                  _last3(128), _last3(128), _last3(128),
                  _full((1, 128)), _full((1, 128)), _full((1, 128)),
                  _full((128, 1)), _full((1, 1)), _full((128, 1)), _full((1, 1))],
        out_specs=_blk3(128),
    )(S, hs, cnt, S, hs, cnt, bcat, gcat, becat, fw1, fb1, fw2, fb2)


def _tc_f(S, hs2, cnt, Wm, bm, gm, bem, Wo, aw1, ab1, aw2, ab2):
    def body(S_ref, hs_ref, cnt_ref, Sl_ref, hsl_ref, cntl_ref,
             wm_ref, bm_ref, gm_ref, bem_ref, wo_ref,
             aw1_ref, ab1_ref, aw2_ref, ab2_ref, out_ref):
        wm = wm_ref[...]
        bm = bm_ref[...]
        gm = gm_ref[...]
        bem = bem_ref[...]

        def xm(Sv, hv, cv):
            dinv = lax.rsqrt(1.0 + cv[:, :, 0:1])
            m = dinv * (Sv + hv)
            x0 = jax.nn.relu(((jnp.dot(m[0], wm) + bm) * _BNS) * gm + bem)
            x1 = jax.nn.relu(((jnp.dot(m[1], wm) + bm) * _BNS) * gm + bem)
            return x0, x1, dinv

        x0, x1, dinv = xm(S_ref[...], hs_ref[...], cnt_ref[...])
        xl0, xl1, _ = xm(Sl_ref[...], hsl_ref[...], cntl_ref[...])
        w1, w2 = _fusion_w(xl0[_LAST:_LAST + 1], xl1[_LAST:_LAST + 1],
                           aw1_ref[...], ab1_ref[...], aw2_ref[...], ab2_ref[...])
        xb = w1 * x0 + w2 * x1
        h = jnp.dot(xb, wo_ref[...])
        out_ref[0] = dinv[0] * h
        out_ref[1] = dinv[1] * h

    return pl.pallas_call(
        body,
        out_shape=jax.ShapeDtypeStruct((_NC, _NP, 128), jnp.float32),
        grid=(_GRID,),
        in_specs=[_blk3(128), _blk3(128), _blk3(128),
                  _last3(128), _last3(128), _last3(128),
                  _full((128, 128)), _full((1, 128)), _full((1, 128)),
                  _full((1, 128)), _full((128, 128)),
                  _full((128, 1)), _full((1, 1)), _full((128, 1)), _full((1, 1))],
        out_specs=_blk3(128),
    )(S, hs2, cnt, S, hs2, cnt, Wm, bm, gm, bem, Wo, aw1, ab1, aw2, ab2)


def _tc_h(S3, hs3, cnt, bo, fw1, fb1, fw2, fb2):
    def body(S_ref, hs_ref, cnt_ref, Sl_ref, hsl_ref, cntl_ref,
             bo_ref, fw1_ref, fb1_ref, fw2_ref, fb2_ref,
             out_ref, g1_ref, g2_ref):
        bo_v = bo_ref[...]

        def gcalc(Sv, hv, cv):
            dinv = lax.rsqrt(1.0 + cv[:, :, 0:1])
            return (dinv * (Sv + hv))[:, :, :48] + bo_v

        gb = gcalc(S_ref[...], hs_ref[...], cnt_ref[...])
        gl = gcalc(Sl_ref[...], hsl_ref[...], cntl_ref[...])
        w1, w2 = _fusion_w(gl[0, _LAST:_LAST + 1], gl[1, _LAST:_LAST + 1],
                           fw1_ref[...], fb1_ref[...], fw2_ref[...], fb2_ref[...])
        g1_ref[...] = gb[0]
        g2_ref[...] = gb[1]
        out_ref[...] = w1 * gb[0] + w2 * gb[1]

    blk2 = pl.BlockSpec((_BLK, 48), lambda i: (i, 0))
    return pl.pallas_call(
        body,
        out_shape=[jax.ShapeDtypeStruct((_NP, 48), jnp.float32)] * 3,
        grid=(_GRID,),
        in_specs=[_blk3(128), _blk3(128), _blk3(128),
                  _last3(128), _last3(128), _last3(128),
                  _full((1, 48)),
                  _full((48, 1)), _full((1, 1)), _full((48, 1)), _full((1, 1))],
        out_specs=[blk2, blk2, blk2],
    )(S3, hs3, cnt, S3, hs3, cnt, bo, fw1, fb1, fw2, fb2)


def kernel(x_list1, x_list2, adj1, adj2, W_init, b_init, g_init, be_init,
           W_mid, b_mid, g_mid, be_mid, W_out, b_out, fc1w1, fc1b1, fc1w2,
           fc1b2, awsw1, awsb1, awsw2, awsb2, fcw1, fcb1, fcw2, fcb2):
    i32 = jnp.int32
    f32 = jnp.float32

    x1p = jnp.pad(x_list1, ((0, 0), (0, _NP - _N), (0, 0)))
    x2p = jnp.pad(x_list2, ((0, 0), (0, _NP - _N), (0, 0)))

    s1, d1 = adj1[0], adj1[1]
    s2, d2 = adj2[0], adj2[1]
    padv = jnp.full((_EP - _E,), _N, i32)
    sg = jnp.concatenate([s1, padv, s2 + _NP, padv + _NP])
    sg = sg.reshape(_NC * _NS * _CPT, 128)
    dg = jnp.concatenate([d1, padv, d2, padv]).reshape(_NC * _NS * _CPT, 128)

    agg128 = _make_agg(128)

    cnt = _make_deg()(dg)

    hsB = _tc_b(x1p, x2p, W_init, cnt)
    S1 = agg128(hsB.reshape(_NC * _NP, 128), sg, dg)
    hs2 = _tc_d(S1, hsB, cnt,
                b_init.reshape(1, 128), g_init.reshape(1, 128),
                be_init.reshape(1, 128),
                fc1w1, fc1b1.reshape(1, 1), fc1w2, fc1b2.reshape(1, 1))

    S2 = agg128(hs2.reshape(_NC * _NP, 128), sg, dg)
    Wo_pad = jnp.pad(W_out, ((0, 0), (0, 88)))
    hs3 = _tc_f(S2, hs2, cnt, W_mid,
                b_mid.reshape(1, 128), g_mid.reshape(1, 128),
                be_mid.reshape(1, 128), Wo_pad,
                awsw1, awsb1.reshape(1, 1), awsw2, awsb2.reshape(1, 1))

    S3 = agg128(hs3.reshape(_NC * _NP, 128), sg, dg)
    bo = jnp.pad(b_out, (0, 8)).reshape(1, 48)
    fw1p = jnp.pad(fcw1, ((0, 8), (0, 0)))
    fw2p = jnp.pad(fcw2, ((0, 8), (0, 0)))
    out, g1, g2 = _tc_h(S3, hs3, cnt, bo,
                        fw1p, fcb1.reshape(1, 1), fw2p, fcb2.reshape(1, 1))
    return (out[:_N, :40], g1[:_N, :40], g2[:_N, :40])


# speedup vs baseline: 12.3920x; 1.2715x over previous
import functools

import jax
import jax.numpy as jnp
from jax import lax
from jax.experimental import pallas as pl
from jax.experimental.pallas import tpu as pltpu
from jax.experimental.pallas import tpu_sc as plsc

_N = 10000
_NP = 10112
_E = 320000
_NC = 2
_NS = 16
_CPT = 160
_EP = _CPT * _NS * 128
_BLK = 1024
_GRID = (_NP + _BLK - 1) // _BLK
_BNS = float(1.0 / (1.0 + 1e-5) ** 0.5)
_KG = 16
_LAST = 7
_LBI = (_N - 1) // 8


@functools.lru_cache(None)
def _make_agg(F):
    mesh = plsc.VectorSubcoreMesh(
        core_axis_name="c", subcore_axis_name="s",
        num_cores=_NC, num_subcores=_NS)
    GQ = _CPT // _KG
    stripe = _NP // _NS

    @functools.partial(
        pl.kernel,
        out_type=jax.ShapeDtypeStruct((_NC, _NP, F), jnp.float32),
        mesh=mesh,
        scratch_types=[
            pltpu.VMEM_SHARED((_NP, F), jnp.float32),
            pltpu.VMEM((2 * _KG, 128), jnp.int32),
            pltpu.VMEM((2 * _KG, 128), jnp.int32),
            pltpu.VMEM((2, 128, F), jnp.float32),
            pltpu.SemaphoreType.DMA,
            pltpu.SemaphoreType.DMA,
            pltpu.SemaphoreType.DMA,
            pltpu.SemaphoreType.DMA,
            pltpu.SemaphoreType.DMA,
        ],
    )
    def agg(hs_hbm, comb_hbm, out_hbm, acc, idx_a, idx_b, rows_v,
            gsem0, gsem1, ssem0, ssem1, isem):
        cid = lax.axis_index("c")
        sid = lax.axis_index("s")
        gsem = (gsem0, gsem1)
        ssem = (ssem0, ssem1)
        base = sid * stripe
        nfull = stripe // 128
        rem_rows = stripe - nfull * 128

        zv = jnp.zeros((16,), jnp.float32)

        def zrow(r, carry):
            for c in range(F // 16):
                rows_v[0, r, pl.ds(c * 16, 16)] = zv
            return carry
        lax.fori_loop(0, 128, zrow, 0)

        zd = [pltpu.async_copy(rows_v.at[0],
                               acc.at[pl.ds(base + k * 128, 128)], gsem0)
              for k in range(nfull)]
        if rem_rows:
            zd.append(pltpu.async_copy(
                rows_v.at[0, pl.ds(0, rem_rows)],
                acc.at[pl.ds(base + nfull * 128, rem_rows)], gsem0))
        for d in zd:
            d.wait()

        grow0 = (cid * _NS + sid) * GQ
        pltpu.sync_copy(comb_hbm.at[grow0], idx_a)
        plsc.subcore_barrier()

        def wait_scatter(b2):
            pltpu.make_async_copy(
                rows_v.at[b2], acc.at[idx_a.at[_KG]], ssem[b2]).wait()

        def wait_gather(b2):
            pltpu.make_async_copy(
                hs_hbm.at[idx_a.at[0]], rows_v.at[b2], gsem[b2]).wait()

        def wait_idx():
            pltpu.make_async_copy(comb_hbm.at[grow0], idx_a, isem).wait()

        def group_body_sync(gq2, carry):
            pltpu.sync_copy(comb_hbm.at[grow0 + gq2], idx_a)
            for b in range(_KG):
                pltpu.async_copy(hs_hbm.at[idx_a.at[b]],
                                 rows_v.at[0], gsem0).wait()
                pltpu.sync_copy(rows_v.at[0], acc.at[idx_a.at[_KG + b]],
                                add=True)
            return carry

        def one_group(t, half, cur, nxt):
            for b in range(_KG):
                b2 = b % 2
                ob = 1 - b2
                if half == 0 and b == 0:
                    @pl.when(t > 0)
                    def _():
                        wait_scatter(ob)
                else:
                    wait_scatter(ob)
                if b < _KG - 1:
                    pltpu.async_copy(hs_hbm.at[cur.at[b + 1]],
                                     rows_v.at[ob], gsem[ob])
                elif half == 0:
                    wait_idx()
                    pltpu.async_copy(hs_hbm.at[nxt.at[0]],
                                     rows_v.at[ob], gsem[ob])
                else:
                    @pl.when(t < GQ // 2 - 1)
                    def _():
                        wait_idx()
                        pltpu.async_copy(hs_hbm.at[nxt.at[0]],
                                         rows_v.at[ob], gsem[ob])
                wait_gather(b2)
                pltpu.async_copy(rows_v.at[b2], acc.at[cur.at[_KG + b]],
                                 ssem[b2], add=True)
                if b == 0:
                    if half == 0:
                        pltpu.async_copy(comb_hbm.at[grow0 + 2 * t + 1],
                                         nxt, isem)
                    else:
                        @pl.when(t < GQ // 2 - 1)
                        def _():
                            pltpu.async_copy(comb_hbm.at[grow0 + 2 * t + 2],
                                             nxt, isem)

        def pair_body(t, carry):
            one_group(t, 0, idx_a, idx_b)
            one_group(t, 1, idx_b, idx_a)
            return carry
        pltpu.async_copy(hs_hbm.at[idx_a.at[0]], rows_v.at[0], gsem0)
        lax.fori_loop(0, GQ // 2, pair_body, 0)
        wait_scatter(1)
        plsc.subcore_barrier()

        wd = [pltpu.async_copy(acc.at[pl.ds(base + k * 128, 128)],
                               out_hbm.at[cid, pl.ds(base + k * 128, 128)],
                               gsem0)
              for k in range(nfull)]
        if rem_rows:
            wd.append(pltpu.async_copy(
                acc.at[pl.ds(base + nfull * 128, rem_rows)],
                out_hbm.at[cid, pl.ds(base + nfull * 128, rem_rows)], gsem0))
        for d in wd:
            d.wait()

    return agg


@functools.lru_cache(None)
def _make_deg():
    F = 128
    mesh = plsc.VectorSubcoreMesh(
        core_axis_name="c", subcore_axis_name="s",
        num_cores=_NC, num_subcores=_NS)
    GQ = _CPT // _KG
    stripe = _NP // _NS

    @functools.partial(
        pl.kernel,
        out_type=jax.ShapeDtypeStruct((_NC, _NP, F), jnp.float32),
        mesh=mesh,
        scratch_types=[
            pltpu.VMEM_SHARED((_NP, F), jnp.float32),
            pltpu.VMEM((_KG, 128), jnp.int32),
            pltpu.VMEM((128, F), jnp.float32),
            pltpu.VMEM((128, F), jnp.float32),
            pltpu.SemaphoreType.DMA,
            pltpu.SemaphoreType.DMA,
        ],
    )
    def deg(didx_hbm, out_hbm, acc, didx_v, ones_v, zero_v, ssem, wsem):
        cid = lax.axis_index("c")
        sid = lax.axis_index("s")
        base = sid * stripe
        nfull = stripe // 128
        rem_rows = stripe - nfull * 128
        zv = jnp.zeros((16,), jnp.float32)
        ov = jnp.ones((16,), jnp.float32)

        def fill_body(r, carry):
            for c in range(F // 16):
                zero_v[r, pl.ds(c * 16, 16)] = zv
                ones_v[r, pl.ds(c * 16, 16)] = ov
            return carry
        lax.fori_loop(0, 128, fill_body, 0)

        zd = [pltpu.async_copy(zero_v,
                               acc.at[pl.ds(base + k * 128, 128)], wsem)
              for k in range(nfull)]
        if rem_rows:
            zd.append(pltpu.async_copy(
                zero_v.at[pl.ds(0, rem_rows)],
                acc.at[pl.ds(base + nfull * 128, rem_rows)], wsem))
        for d in zd:
            d.wait()

        row0 = (cid * _NS + sid) * GQ
        plsc.subcore_barrier()

        def wait_scatter():
            pltpu.make_async_copy(
                ones_v, acc.at[didx_v.at[0]], ssem).wait()

        def group_body(gq, carry):
            pltpu.sync_copy(didx_hbm.at[row0 + gq], didx_v)
            for b in range(_KG):
                pltpu.async_copy(ones_v, acc.at[didx_v.at[b]], ssem,
                                 add=True)
            for b in range(_KG):
                wait_scatter()
            return carry
        lax.fori_loop(0, GQ, group_body, 0)
        plsc.subcore_barrier()

        wd = [pltpu.async_copy(acc.at[pl.ds(base + k * 128, 128)],
                               out_hbm.at[cid, pl.ds(base + k * 128, 128)],
                               wsem)
              for k in range(nfull)]
        if rem_rows:
            wd.append(pltpu.async_copy(
                acc.at[pl.ds(base + nfull * 128, rem_rows)],
                out_hbm.at[cid, pl.ds(base + nfull * 128, rem_rows)], wsem))
        for d in wd:
            d.wait()

    return deg


def _blk3(w):
    return pl.BlockSpec((2, _BLK, w), lambda i: (0, i, 0))


def _last3(w):
    lbi = (_N - 1) // 8
    return pl.BlockSpec((2, 8, w), lambda i, _l=lbi: (0, _l, 0))


def _full(shape):
    nd = len(shape)
    return pl.BlockSpec(shape, lambda i, _n=nd: (0,) * _n)


def _fusion_w(a_row, b_row, wa, ba, wb, bb):
    la = jax.nn.sigmoid(jnp.dot(a_row, wa) + ba)
    lb = jax.nn.sigmoid(jnp.dot(b_row, wb) + bb)
    s = jnp.abs(la) + jnp.abs(lb)
    return la / s, lb / s


def _tc_b(x1, x2, W, cnt):
    def body(x1_ref, x2_ref, w_ref, cnt_ref, out_ref):
        dinv = lax.rsqrt(1.0 + cnt_ref[:, :, 0:1])
        w0 = w_ref[0]
        w1 = w_ref[1]
        h1 = jnp.concatenate(
            [jnp.dot(x1_ref[0], w0), jnp.dot(x1_ref[1], w1)], axis=-1)
        h2 = jnp.concatenate(
            [jnp.dot(x2_ref[0], w0), jnp.dot(x2_ref[1], w1)], axis=-1)
        out_ref[0] = dinv[0] * h1
        out_ref[1] = dinv[1] * h2

    return pl.pallas_call(
        body,
        out_shape=jax.ShapeDtypeStruct((_NC, _NP, 128), jnp.float32),
        grid=(_GRID,),
        in_specs=[_blk3(128), _blk3(128), _full((2, 128, 64)), _blk3(128)],
        out_specs=_blk3(128),
    )(x1, x2, W, cnt)


def _tc_d(S, hs, cnt, bcat, gcat, becat, fw1, fb1, fw2, fb2):
    def body(S_ref, hs_ref, cnt_ref, Sl_ref, hsl_ref, cntl_ref,
             b_ref, g_ref, be_ref, fw1_ref, fb1_ref, fw2_ref, fb2_ref,
             out_ref):
        b = b_ref[...]
        g = g_ref[...]
        be = be_ref[...]

        def xcalc(Sv, hv, cv):
            dinv = lax.rsqrt(1.0 + cv[:, :, 0:1])
            m = dinv * (Sv + hv) + b
            return jax.nn.relu((m * _BNS) * g + be), dinv

        x_blk, dinv = xcalc(S_ref[...], hs_ref[...], cnt_ref[...])
        x_last, _ = xcalc(Sl_ref[...], hsl_ref[...], cntl_ref[...])
        w1, w2 = _fusion_w(x_last[0, _LAST:_LAST + 1], x_last[1, _LAST:_LAST + 1],
                           fw1_ref[...], fb1_ref[...], fw2_ref[...], fb2_ref[...])
        xa = w1 * x_blk[0] + w2 * x_blk[1]
        out_ref[0] = dinv[0] * xa
        out_ref[1] = dinv[1] * xa

    return pl.pallas_call(
        body,
        out_shape=jax.ShapeDtypeStruct((_NC, _NP, 128), jnp.float32),
        grid=(_GRID,),
        in_specs=[_blk3(128), _blk3(128), _blk3(128),
                  _last3(128), _last3(128), _last3(128),
                  _full((1, 128)), _full((1, 128)), _full((1, 128)),
                  _full((128, 1)), _full((1, 1)), _full((128, 1)), _full((1, 1))],
        out_specs=_blk3(128),
    )(S, hs, cnt, S, hs, cnt, bcat, gcat, becat, fw1, fb1, fw2, fb2)


def _tc_f(S, hs2, cnt, Wm, bm, gm, bem, Wo, aw1, ab1, aw2, ab2):
    def body(S_ref, hs_ref, cnt_ref, Sl_ref, hsl_ref, cntl_ref,
             wm_ref, bm_ref, gm_ref, bem_ref, wo_ref,
             aw1_ref, ab1_ref, aw2_ref, ab2_ref, out_ref):
        wm = wm_ref[...]
        bm = bm_ref[...]
        gm = gm_ref[...]
        bem = bem_ref[...]

        def xm(Sv, hv, cv):
            dinv = lax.rsqrt(1.0 + cv[:, :, 0:1])
            m = dinv * (Sv + hv)
            x0 = jax.nn.relu(((jnp.dot(m[0], wm) + bm) * _BNS) * gm + bem)
            x1 = jax.nn.relu(((jnp.dot(m[1], wm) + bm) * _BNS) * gm + bem)
            return x0, x1, dinv

        x0, x1, dinv = xm(S_ref[...], hs_ref[...], cnt_ref[...])
        xl0, xl1, _ = xm(Sl_ref[...], hsl_ref[...], cntl_ref[...])
        w1, w2 = _fusion_w(xl0[_LAST:_LAST + 1], xl1[_LAST:_LAST + 1],
                           aw1_ref[...], ab1_ref[...], aw2_ref[...], ab2_ref[...])
        xb = w1 * x0 + w2 * x1
        h = jnp.dot(xb, wo_ref[...])
        out_ref[0] = dinv[0] * h
        out_ref[1] = dinv[1] * h

    return pl.pallas_call(
        body,
        out_shape=jax.ShapeDtypeStruct((_NC, _NP, 128), jnp.float32),
        grid=(_GRID,),
        in_specs=[_blk3(128), _blk3(128), _blk3(128),
                  _last3(128), _last3(128), _last3(128),
                  _full((128, 128)), _full((1, 128)), _full((1, 128)),
                  _full((1, 128)), _full((128, 128)),
                  _full((128, 1)), _full((1, 1)), _full((128, 1)), _full((1, 1))],
        out_specs=_blk3(128),
    )(S, hs2, cnt, S, hs2, cnt, Wm, bm, gm, bem, Wo, aw1, ab1, aw2, ab2)


def _tc_h(S3, hs3, cnt, bo, fw1, fb1, fw2, fb2):
    def body(S_ref, hs_ref, cnt_ref, Sl_ref, hsl_ref, cntl_ref,
             bo_ref, fw1_ref, fb1_ref, fw2_ref, fb2_ref,
             out_ref, g1_ref, g2_ref):
        bo_v = bo_ref[...]

        def gcalc(Sv, hv, cv):
            dinv = lax.rsqrt(1.0 + cv[:, :, 0:1])
            return (dinv * (Sv + hv))[:, :, :48] + bo_v

        gb = gcalc(S_ref[...], hs_ref[...], cnt_ref[...])
        gl = gcalc(Sl_ref[...], hsl_ref[...], cntl_ref[...])
        w1, w2 = _fusion_w(gl[0, _LAST:_LAST + 1], gl[1, _LAST:_LAST + 1],
                           fw1_ref[...], fb1_ref[...], fw2_ref[...], fb2_ref[...])
        g1_ref[...] = gb[0]
        g2_ref[...] = gb[1]
        out_ref[...] = w1 * gb[0] + w2 * gb[1]

    blk2 = pl.BlockSpec((_BLK, 48), lambda i: (i, 0))
    return pl.pallas_call(
        body,
        out_shape=[jax.ShapeDtypeStruct((_NP, 48), jnp.float32)] * 3,
        grid=(_GRID,),
        in_specs=[_blk3(128), _blk3(128), _blk3(128),
                  _last3(128), _last3(128), _last3(128),
                  _full((1, 48)),
                  _full((48, 1)), _full((1, 1)), _full((48, 1)), _full((1, 1))],
        out_specs=[blk2, blk2, blk2],
    )(S3, hs3, cnt, S3, hs3, cnt, bo, fw1, fb1, fw2, fb2)


def kernel(x_list1, x_list2, adj1, adj2, W_init, b_init, g_init, be_init,
           W_mid, b_mid, g_mid, be_mid, W_out, b_out, fc1w1, fc1b1, fc1w2,
           fc1b2, awsw1, awsb1, awsw2, awsb2, fcw1, fcb1, fcw2, fcb2):
    i32 = jnp.int32
    f32 = jnp.float32

    x1p = jnp.pad(x_list1, ((0, 0), (0, _NP - _N), (0, 0)))
    x2p = jnp.pad(x_list2, ((0, 0), (0, _NP - _N), (0, 0)))

    s1, d1 = adj1[0], adj1[1]
    s2, d2 = adj2[0], adj2[1]
    padv = jnp.full((_EP - _E,), _N, i32)
    gq = _CPT // _KG
    sg = jnp.concatenate([s1, padv, s2 + _NP, padv + _NP])
    sgr = sg.reshape(_NC * _NS, gq, _KG, 128)
    dgr = jnp.concatenate([d1, padv, d2, padv]).reshape(
        _NC * _NS, gq, _KG, 128)
    comb = jnp.concatenate([sgr, dgr], axis=2).reshape(
        _NC * _NS * gq, 2 * _KG, 128)
    dgc = dgr.reshape(_NC * _NS * gq, _KG, 128)

    agg128 = _make_agg(128)

    cnt = _make_deg()(dgc)

    hsB = _tc_b(x1p, x2p, W_init, cnt)
    S1 = agg128(hsB.reshape(_NC * _NP, 128), comb)
    hs2 = _tc_d(S1, hsB, cnt,
                b_init.reshape(1, 128), g_init.reshape(1, 128),
                be_init.reshape(1, 128),
                fc1w1, fc1b1.reshape(1, 1), fc1w2, fc1b2.reshape(1, 1))

    S2 = agg128(hs2.reshape(_NC * _NP, 128), comb)
    Wo_pad = jnp.pad(W_out, ((0, 0), (0, 88)))
    hs3 = _tc_f(S2, hs2, cnt, W_mid,
                b_mid.reshape(1, 128), g_mid.reshape(1, 128),
                be_mid.reshape(1, 128), Wo_pad,
                awsw1, awsb1.reshape(1, 1), awsw2, awsb2.reshape(1, 1))

    S3 = agg128(hs3.reshape(_NC * _NP, 128), comb)
    bo = jnp.pad(b_out, (0, 8)).reshape(1, 48)
    fw1p = jnp.pad(fcw1, ((0, 8), (0, 0)))
    fw2p = jnp.pad(fcw2, ((0, 8), (0, 0)))
    out, g1, g2 = _tc_h(S3, hs3, cnt, bo,
                        fw1p, fcb1.reshape(1, 1), fw2p, fcb2.reshape(1, 1))
    return (out[:_N, :40], g1[:_N, :40], g2[:_N, :40])


# speedup vs baseline: 24.7543x; 1.9976x over previous
import functools

import jax
import jax.numpy as jnp
from jax import lax
from jax.experimental import pallas as pl
from jax.experimental.pallas import tpu as pltpu
from jax.experimental.pallas import tpu_sc as plsc

_N = 10000
_NP = 10112
_E = 320000
_NC = 2
_NS = 16
_CPT = 160
_EP = _CPT * _NS * 128
_BLK = 1024
_GRID = (_NP + _BLK - 1) // _BLK
_BNS = float(1.0 / (1.0 + 1e-5) ** 0.5)
_KG = 16
_LAST = 7
_LBI = (_N - 1) // 8


@functools.lru_cache(None)
def _make_agg(npass):
    mesh = plsc.VectorSubcoreMesh(
        core_axis_name="c", subcore_axis_name="s",
        num_cores=_NC, num_subcores=_NS)
    GQ = _CPT // _KG
    stripe = _NP // _NS
    FH = 64

    @functools.partial(
        pl.kernel,
        out_type=jax.ShapeDtypeStruct((_NC, npass, _NP, FH), jnp.float32),
        mesh=mesh,
        scratch_types=[
            pltpu.VMEM_SHARED((_NP, FH), jnp.float32),
            pltpu.VMEM_SHARED((_NP, FH), jnp.float32),
            pltpu.VMEM((2 * _KG, 128), jnp.int32),
            pltpu.VMEM((2 * _KG, 128), jnp.int32),
            pltpu.VMEM((2, 128, FH), jnp.float32),
            pltpu.SemaphoreType.DMA,
            pltpu.SemaphoreType.DMA,
            pltpu.SemaphoreType.DMA,
            pltpu.SemaphoreType.DMA,
            pltpu.SemaphoreType.DMA,
        ],
    )
    def agg(hs_hbm, comb_hbm, out_hbm, table, acc, idx_a, idx_b, rows_v,
            gsem0, gsem1, ssem0, ssem1, isem):
        cid = lax.axis_index("c")
        sid = lax.axis_index("s")
        gsem = (gsem0, gsem1)
        ssem = (ssem0, ssem1)
        base = sid * stripe
        nfull = stripe // 128
        rem_rows = stripe - nfull * 128
        grow0 = (cid * _NS + sid) * GQ
        zv = jnp.zeros((16,), jnp.float32)

        def wait_scatter(b2):
            pltpu.make_async_copy(
                rows_v.at[b2], acc.at[idx_a.at[_KG]], ssem[b2]).wait()

        def wait_gather(b2):
            pltpu.make_async_copy(
                table.at[idx_a.at[0]], rows_v.at[b2], gsem[b2]).wait()

        def wait_idx():
            pltpu.make_async_copy(comb_hbm.at[grow0], idx_a, isem).wait()

        def one_group(t, half, cur, nxt):
            for b in range(_KG):
                b2 = b % 2
                ob = 1 - b2
                if half == 0 and b == 0:
                    @pl.when(t > 0)
                    def _():
                        wait_scatter(ob)
                else:
                    wait_scatter(ob)
                if b < _KG - 1:
                    pltpu.async_copy(table.at[cur.at[b + 1]],
                                     rows_v.at[ob], gsem[ob])
                elif half == 0:
                    wait_idx()
                    pltpu.async_copy(table.at[nxt.at[0]],
                                     rows_v.at[ob], gsem[ob])
                else:
                    @pl.when(t < GQ // 2 - 1)
                    def _():
                        wait_idx()
                        pltpu.async_copy(table.at[nxt.at[0]],
                                         rows_v.at[ob], gsem[ob])
                wait_gather(b2)
                pltpu.async_copy(rows_v.at[b2], acc.at[cur.at[_KG + b]],
                                 ssem[b2], add=True)
                if b == 0:
                    if half == 0:
                        pltpu.async_copy(comb_hbm.at[grow0 + 2 * t + 1],
                                         nxt, isem)
                    else:
                        @pl.when(t < GQ // 2 - 1)
                        def _():
                            pltpu.async_copy(comb_hbm.at[grow0 + 2 * t + 2],
                                             nxt, isem)

        def pair_body(t, carry):
            one_group(t, 0, idx_a, idx_b)
            one_group(t, 1, idx_b, idx_a)
            return carry

        for h in range(npass):
            def zrow(r, carry):
                for c in range(FH // 16):
                    rows_v[0, r, pl.ds(c * 16, 16)] = zv
                return carry
            lax.fori_loop(0, 128, zrow, 0)

            pd = []
            for k in range(nfull):
                pd.append(pltpu.async_copy(
                    hs_hbm.at[cid, h, pl.ds(base + k * 128, 128)],
                    table.at[pl.ds(base + k * 128, 128)], gsem1))
                pd.append(pltpu.async_copy(
                    rows_v.at[0],
                    acc.at[pl.ds(base + k * 128, 128)], ssem0))
            if rem_rows:
                pd.append(pltpu.async_copy(
                    hs_hbm.at[cid, h, pl.ds(base + nfull * 128, rem_rows)],
                    table.at[pl.ds(base + nfull * 128, rem_rows)], gsem1))
                pd.append(pltpu.async_copy(
                    rows_v.at[0, pl.ds(0, rem_rows)],
                    acc.at[pl.ds(base + nfull * 128, rem_rows)], ssem0))
            for d in pd:
                d.wait()
            pltpu.sync_copy(comb_hbm.at[grow0], idx_a)
            plsc.subcore_barrier()

            pltpu.async_copy(table.at[idx_a.at[0]], rows_v.at[0], gsem0)
            lax.fori_loop(0, GQ // 2, pair_body, 0)
            wait_scatter(1)
            plsc.subcore_barrier()

            wd = [pltpu.async_copy(
                acc.at[pl.ds(base + k * 128, 128)],
                out_hbm.at[cid, h, pl.ds(base + k * 128, 128)], gsem0)
                for k in range(nfull)]
            if rem_rows:
                wd.append(pltpu.async_copy(
                    acc.at[pl.ds(base + nfull * 128, rem_rows)],
                    out_hbm.at[cid, h, pl.ds(base + nfull * 128, rem_rows)],
                    gsem0))
            for d in wd:
                d.wait()
            if h + 1 < npass:
                plsc.subcore_barrier()

    return agg


@functools.lru_cache(None)
def _make_deg():
    F = 64
    mesh = plsc.VectorSubcoreMesh(
        core_axis_name="c", subcore_axis_name="s",
        num_cores=_NC, num_subcores=_NS)
    GQ = _CPT // _KG
    stripe = _NP // _NS

    @functools.partial(
        pl.kernel,
        out_type=jax.ShapeDtypeStruct((_NC, _NP, F), jnp.float32),
        mesh=mesh,
        scratch_types=[
            pltpu.VMEM_SHARED((_NP, F), jnp.float32),
            pltpu.VMEM((_KG, 128), jnp.int32),
            pltpu.VMEM((128, F), jnp.float32),
            pltpu.VMEM((128, F), jnp.float32),
            pltpu.SemaphoreType.DMA,
            pltpu.SemaphoreType.DMA,
        ],
    )
    def deg(didx_hbm, out_hbm, acc, didx_v, ones_v, zero_v, ssem, wsem):
        cid = lax.axis_index("c")
        sid = lax.axis_index("s")
        base = sid * stripe
        nfull = stripe // 128
        rem_rows = stripe - nfull * 128
        zv = jnp.zeros((16,), jnp.float32)
        ov = jnp.ones((16,), jnp.float32)

        def fill_body(r, carry):
            for c in range(F // 16):
                zero_v[r, pl.ds(c * 16, 16)] = zv
                ones_v[r, pl.ds(c * 16, 16)] = ov
            return carry
        lax.fori_loop(0, 128, fill_body, 0)

        zd = [pltpu.async_copy(zero_v,
                               acc.at[pl.ds(base + k * 128, 128)], wsem)
              for k in range(nfull)]
        if rem_rows:
            zd.append(pltpu.async_copy(
                zero_v.at[pl.ds(0, rem_rows)],
                acc.at[pl.ds(base + nfull * 128, rem_rows)], wsem))
        for d in zd:
            d.wait()

        row0 = (cid * _NS + sid) * GQ
        plsc.subcore_barrier()

        def wait_scatter():
            pltpu.make_async_copy(
                ones_v, acc.at[didx_v.at[0]], ssem).wait()

        def group_body(gq, carry):
            pltpu.sync_copy(didx_hbm.at[row0 + gq], didx_v)
            for b in range(_KG):
                pltpu.async_copy(ones_v, acc.at[didx_v.at[b]], ssem,
                                 add=True)
            for b in range(_KG):
                wait_scatter()
            return carry
        lax.fori_loop(0, GQ, group_body, 0)
        plsc.subcore_barrier()

        wd = [pltpu.async_copy(acc.at[pl.ds(base + k * 128, 128)],
                               out_hbm.at[cid, pl.ds(base + k * 128, 128)],
                               wsem)
              for k in range(nfull)]
        if rem_rows:
            wd.append(pltpu.async_copy(
                acc.at[pl.ds(base + nfull * 128, rem_rows)],
                out_hbm.at[cid, pl.ds(base + nfull * 128, rem_rows)], wsem))
        for d in wd:
            d.wait()

    return deg


def _blk3(w):
    return pl.BlockSpec((2, _BLK, w), lambda i: (0, i, 0))


def _blk4():
    return pl.BlockSpec((2, 2, _BLK, 64), lambda i: (0, 0, i, 0))


def _last4():
    return pl.BlockSpec((2, 2, 8, 64), lambda i, _l=_LBI: (0, 0, _l, 0))


def _last3(w):
    lbi = (_N - 1) // 8
    return pl.BlockSpec((2, 8, w), lambda i, _l=lbi: (0, _l, 0))


def _full(shape):
    nd = len(shape)
    return pl.BlockSpec(shape, lambda i, _n=nd: (0,) * _n)


def _fusion_w(a_row, b_row, wa, ba, wb, bb):
    la = jax.nn.sigmoid(jnp.dot(a_row, wa) + ba)
    lb = jax.nn.sigmoid(jnp.dot(b_row, wb) + bb)
    s = jnp.abs(la) + jnp.abs(lb)
    return la / s, lb / s


def _tc_b(x1, x2, W, cnt):
    def body(x1_ref, x2_ref, w_ref, cnt_ref, out_ref):
        dinv = lax.rsqrt(1.0 + cnt_ref[:, :, 0:1])
        w0 = w_ref[0]
        w1 = w_ref[1]
        h1 = jnp.concatenate(
            [jnp.dot(x1_ref[0], w0), jnp.dot(x1_ref[1], w1)], axis=-1)
        h2 = jnp.concatenate(
            [jnp.dot(x2_ref[0], w0), jnp.dot(x2_ref[1], w1)], axis=-1)
        v1 = dinv[0] * h1
        v2 = dinv[1] * h2
        out_ref[0, 0] = v1[:, :64]
        out_ref[0, 1] = v1[:, 64:]
        out_ref[1, 0] = v2[:, :64]
        out_ref[1, 1] = v2[:, 64:]

    return pl.pallas_call(
        body,
        out_shape=jax.ShapeDtypeStruct((_NC, 2, _NP, 64), jnp.float32),
        grid=(_GRID,),
        in_specs=[_blk3(128), _blk3(128), _full((2, 128, 64)), _blk3(64)],
        out_specs=_blk4(),
    )(x1, x2, W, cnt)


def _cat(v):
    return jnp.concatenate([v[:, 0], v[:, 1]], axis=-1)


def _tc_d(S, hs, cnt, bcat, gcat, becat, fw1, fb1, fw2, fb2):
    def body(S_ref, hs_ref, cnt_ref, Sl_ref, hsl_ref, cntl_ref,
             b_ref, g_ref, be_ref, fw1_ref, fb1_ref, fw2_ref, fb2_ref,
             out_ref):
        b = b_ref[...]
        g = g_ref[...]
        be = be_ref[...]

        def xcalc(Sv, hv, cv):
            dinv = lax.rsqrt(1.0 + cv[:, :, 0:1])
            m = dinv * (Sv + hv) + b
            return jax.nn.relu((m * _BNS) * g + be), dinv

        x_blk, dinv = xcalc(_cat(S_ref[...]), _cat(hs_ref[...]), cnt_ref[...])
        x_last, _ = xcalc(_cat(Sl_ref[...]), _cat(hsl_ref[...]), cntl_ref[...])
        w1, w2 = _fusion_w(x_last[0, _LAST:_LAST + 1], x_last[1, _LAST:_LAST + 1],
                           fw1_ref[...], fb1_ref[...], fw2_ref[...], fb2_ref[...])
        xa = w1 * x_blk[0] + w2 * x_blk[1]
        v1 = dinv[0] * xa
        v2 = dinv[1] * xa
        out_ref[0, 0] = v1[:, :64]
        out_ref[0, 1] = v1[:, 64:]
        out_ref[1, 0] = v2[:, :64]
        out_ref[1, 1] = v2[:, 64:]

    return pl.pallas_call(
        body,
        out_shape=jax.ShapeDtypeStruct((_NC, 2, _NP, 64), jnp.float32),
        grid=(_GRID,),
        in_specs=[_blk4(), _blk4(), _blk3(64),
                  _last4(), _last4(), _last3(64),
                  _full((1, 128)), _full((1, 128)), _full((1, 128)),
                  _full((128, 1)), _full((1, 1)), _full((128, 1)), _full((1, 1))],
        out_specs=_blk4(),
    )(S, hs, cnt, S, hs, cnt, bcat, gcat, becat, fw1, fb1, fw2, fb2)


def _tc_f(S, hs2, cnt, Wm, bm, gm, bem, Wo, aw1, ab1, aw2, ab2):
    def body(S_ref, hs_ref, cnt_ref, Sl_ref, hsl_ref, cntl_ref,
             wm_ref, bm_ref, gm_ref, bem_ref, wo_ref,
             aw1_ref, ab1_ref, aw2_ref, ab2_ref, out_ref):
        wm = wm_ref[...]
        bm = bm_ref[...]
        gm = gm_ref[...]
        bem = bem_ref[...]

        def xm(Sv, hv, cv):
            dinv = lax.rsqrt(1.0 + cv[:, :, 0:1])
            m = dinv * (Sv + hv)
            x0 = jax.nn.relu(((jnp.dot(m[0], wm) + bm) * _BNS) * gm + bem)
            x1 = jax.nn.relu(((jnp.dot(m[1], wm) + bm) * _BNS) * gm + bem)
            return x0, x1, dinv

        x0, x1, dinv = xm(_cat(S_ref[...]), _cat(hs_ref[...]), cnt_ref[...])
        xl0, xl1, _ = xm(_cat(Sl_ref[...]), _cat(hsl_ref[...]), cntl_ref[...])
        w1, w2 = _fusion_w(xl0[_LAST:_LAST + 1], xl1[_LAST:_LAST + 1],
                           aw1_ref[...], ab1_ref[...], aw2_ref[...], ab2_ref[...])
        xb = w1 * x0 + w2 * x1
        h = jnp.dot(xb, wo_ref[...])
        v1 = dinv[0] * h
        v2 = dinv[1] * h
        out_ref[0, 0] = v1[:, :64]
        out_ref[0, 1] = v1[:, 64:]
        out_ref[1, 0] = v2[:, :64]
        out_ref[1, 1] = v2[:, 64:]

    return pl.pallas_call(
        body,
        out_shape=jax.ShapeDtypeStruct((_NC, 2, _NP, 64), jnp.float32),
        grid=(_GRID,),
        in_specs=[_blk4(), _blk4(), _blk3(64),
                  _last4(), _last4(), _last3(64),
                  _full((128, 128)), _full((1, 128)), _full((1, 128)),
                  _full((1, 128)), _full((128, 128)),
                  _full((128, 1)), _full((1, 1)), _full((128, 1)), _full((1, 1))],
        out_specs=_blk4(),
    )(S, hs2, cnt, S, hs2, cnt, Wm, bm, gm, bem, Wo, aw1, ab1, aw2, ab2)


def _tc_h(S3, hs3, cnt, bo, fw1, fb1, fw2, fb2):
    def body(S_ref, hs_ref, cnt_ref, Sl_ref, hsl_ref, cntl_ref,
             bo_ref, fw1_ref, fb1_ref, fw2_ref, fb2_ref,
             out_ref, g1_ref, g2_ref):
        bo_v = bo_ref[...]

        def gcalc(Sv, hv, cv):
            dinv = lax.rsqrt(1.0 + cv[:, :, 0:1])
            return (dinv * (Sv[:, 0] + hv[:, 0]))[:, :, :48] + bo_v

        gb = gcalc(S_ref[...], hs_ref[...], cnt_ref[...])
        gl = gcalc(Sl_ref[...], hsl_ref[...], cntl_ref[...])
        w1, w2 = _fusion_w(gl[0, _LAST:_LAST + 1], gl[1, _LAST:_LAST + 1],
                           fw1_ref[...], fb1_ref[...], fw2_ref[...], fb2_ref[...])
        g1_ref[...] = gb[0]
        g2_ref[...] = gb[1]
        out_ref[...] = w1 * gb[0] + w2 * gb[1]

    blk2 = pl.BlockSpec((_BLK, 48), lambda i: (i, 0))
    h1spec = pl.BlockSpec((2, 1, _BLK, 64), lambda i: (0, 0, i, 0))
    h1last = pl.BlockSpec((2, 1, 8, 64), lambda i, _l=_LBI: (0, 0, _l, 0))
    return pl.pallas_call(
        body,
        out_shape=[jax.ShapeDtypeStruct((_NP, 48), jnp.float32)] * 3,
        grid=(_GRID,),
        in_specs=[h1spec, h1spec, _blk3(64),
                  h1last, h1last, _last3(64),
                  _full((1, 48)),
                  _full((48, 1)), _full((1, 1)), _full((48, 1)), _full((1, 1))],
        out_specs=[blk2, blk2, blk2],
    )(S3, hs3, cnt, S3, hs3, cnt, bo, fw1, fb1, fw2, fb2)


def kernel(x_list1, x_list2, adj1, adj2, W_init, b_init, g_init, be_init,
           W_mid, b_mid, g_mid, be_mid, W_out, b_out, fc1w1, fc1b1, fc1w2,
           fc1b2, awsw1, awsb1, awsw2, awsb2, fcw1, fcb1, fcw2, fcb2):
    i32 = jnp.int32

    x1p = jnp.pad(x_list1, ((0, 0), (0, _NP - _N), (0, 0)))
    x2p = jnp.pad(x_list2, ((0, 0), (0, _NP - _N), (0, 0)))

    s1, d1 = adj1[0], adj1[1]
    s2, d2 = adj2[0], adj2[1]
    padv = jnp.full((_EP - _E,), _N, i32)
    gq = _CPT // _KG
    sgr = jnp.concatenate([s1, padv, s2, padv]).reshape(
        _NC * _NS, gq, _KG, 128)
    dgr = jnp.concatenate([d1, padv, d2, padv]).reshape(
        _NC * _NS, gq, _KG, 128)
    comb = jnp.concatenate([sgr, dgr], axis=2).reshape(
        _NC * _NS * gq, 2 * _KG, 128)
    dgc = dgr.reshape(_NC * _NS * gq, _KG, 128)

    agg2p = _make_agg(2)
    agg1p = _make_agg(1)

    cnt = _make_deg()(dgc)

    hsB = _tc_b(x1p, x2p, W_init, cnt)
    S1 = agg2p(hsB, comb)
    hs2 = _tc_d(S1, hsB, cnt,
                b_init.reshape(1, 128), g_init.reshape(1, 128),
                be_init.reshape(1, 128),
                fc1w1, fc1b1.reshape(1, 1), fc1w2, fc1b2.reshape(1, 1))

    S2 = agg2p(hs2, comb)
    Wo_pad = jnp.pad(W_out, ((0, 0), (0, 88)))
    hs3 = _tc_f(S2, hs2, cnt, W_mid,
                b_mid.reshape(1, 128), g_mid.reshape(1, 128),
                be_mid.reshape(1, 128), Wo_pad,
                awsw1, awsb1.reshape(1, 1), awsw2, awsb2.reshape(1, 1))

    S3 = agg1p(hs3, comb)
    bo = jnp.pad(b_out, (0, 8)).reshape(1, 48)
    fw1p = jnp.pad(fcw1, ((0, 8), (0, 0)))
    fw2p = jnp.pad(fcw2, ((0, 8), (0, 0)))
    out, g1, g2 = _tc_h(S3, hs3, cnt, bo,
                        fw1p, fcb1.reshape(1, 1), fw2p, fcb2.reshape(1, 1))
    return (out[:_N, :40], g1[:_N, :40], g2[:_N, :40])


# speedup vs baseline: 26.5509x; 1.0726x over previous
import functools

import jax
import jax.numpy as jnp
from jax import lax
from jax.experimental import pallas as pl
from jax.experimental.pallas import tpu as pltpu
from jax.experimental.pallas import tpu_sc as plsc

_N = 10000
_NP = 10112
_E = 320000
_NC = 2
_NS = 16
_CPT = 160
_EP = _CPT * _NS * 128
_BLK = 1024
_GRID = (_NP + _BLK - 1) // _BLK
_BNS = float(1.0 / (1.0 + 1e-5) ** 0.5)
_KG = 16
_LAST = 7
_LBI = (_N - 1) // 8


@functools.lru_cache(None)
def _make_agg(npass):
    mesh = plsc.VectorSubcoreMesh(
        core_axis_name="c", subcore_axis_name="s",
        num_cores=_NC, num_subcores=_NS)
    GQ = _CPT // _KG
    stripe = _NP // _NS
    FH = 64

    @functools.partial(
        pl.kernel,
        out_type=jax.ShapeDtypeStruct((_NC, npass, _NP, FH), jnp.float32),
        mesh=mesh,
        scratch_types=[
            pltpu.VMEM_SHARED((_NP, FH), jnp.float32),
            pltpu.VMEM_SHARED((_NP, FH), jnp.float32),
            pltpu.VMEM((2 * _KG, 128), jnp.int32),
            pltpu.VMEM((2 * _KG, 128), jnp.int32),
            pltpu.VMEM((4, 128, FH), jnp.float32),
            pltpu.SemaphoreType.DMA,
            pltpu.SemaphoreType.DMA,
            pltpu.SemaphoreType.DMA,
            pltpu.SemaphoreType.DMA,
            pltpu.SemaphoreType.DMA,
            pltpu.SemaphoreType.DMA,
            pltpu.SemaphoreType.DMA,
            pltpu.SemaphoreType.DMA,
            pltpu.SemaphoreType.DMA,
        ],
        compiler_params=pltpu.CompilerParams(use_tc_tiling_on_sc=False),
    )
    def agg(hs_hbm, comb_hbm, out_hbm, table, acc, idx_a, idx_b, rows_v,
            gsem0, gsem1, gsem2, gsem3, ssem0, ssem1, ssem2, ssem3, isem):
        cid = lax.axis_index("c")
        sid = lax.axis_index("s")
        gsem = (gsem0, gsem1, gsem2, gsem3)
        ssem = (ssem0, ssem1, ssem2, ssem3)
        base = sid * stripe
        nfull = stripe // 128
        rem_rows = stripe - nfull * 128
        grow0 = (cid * _NS + sid) * GQ
        zv = jnp.zeros((16,), jnp.float32)

        def wait_scatter(b2):
            pltpu.make_async_copy(
                rows_v.at[b2], acc.at[idx_a.at[_KG]], ssem[b2]).wait()

        def wait_gather(b2):
            pltpu.make_async_copy(
                table.at[idx_a.at[0]], rows_v.at[b2], gsem[b2]).wait()

        def wait_idx():
            pltpu.make_async_copy(comb_hbm.at[grow0], idx_a, isem).wait()

        def one_group(t, half, cur, nxt):
            for b in range(_KG):
                bi = b % 4
                bn = (b + 1) % 4
                if half == 0 and b < 3:
                    @pl.when(t > 0)
                    def _():
                        wait_scatter(bn)
                else:
                    wait_scatter(bn)
                if b < _KG - 1:
                    pltpu.async_copy(table.at[cur.at[b + 1]],
                                     rows_v.at[bn], gsem[bn])
                elif half == 0:
                    wait_idx()
                    pltpu.async_copy(table.at[nxt.at[0]],
                                     rows_v.at[bn], gsem[bn])
                else:
                    @pl.when(t < GQ // 2 - 1)
                    def _():
                        wait_idx()
                        pltpu.async_copy(table.at[nxt.at[0]],
                                         rows_v.at[bn], gsem[bn])
                wait_gather(bi)
                pltpu.async_copy(rows_v.at[bi], acc.at[cur.at[_KG + b]],
                                 ssem[bi], add=True)
                if b == 0:
                    if half == 0:
                        pltpu.async_copy(comb_hbm.at[grow0 + 2 * t + 1],
                                         nxt, isem)
                    else:
                        @pl.when(t < GQ // 2 - 1)
                        def _():
                            pltpu.async_copy(comb_hbm.at[grow0 + 2 * t + 2],
                                             nxt, isem)

        def pair_body(t, carry):
            one_group(t, 0, idx_a, idx_b)
            one_group(t, 1, idx_b, idx_a)
            return carry

        for h in range(npass):
            def zrow(r, carry):
                for c in range(FH // 16):
                    rows_v[0, r, pl.ds(c * 16, 16)] = zv
                return carry
            lax.fori_loop(0, 128, zrow, 0)

            pd = []
            for k in range(nfull):
                pd.append(pltpu.async_copy(
                    hs_hbm.at[cid, h, pl.ds(base + k * 128, 128)],
                    table.at[pl.ds(base + k * 128, 128)], gsem1))
                pd.append(pltpu.async_copy(
                    rows_v.at[0],
                    acc.at[pl.ds(base + k * 128, 128)], ssem0))
            if rem_rows:
                pd.append(pltpu.async_copy(
                    hs_hbm.at[cid, h, pl.ds(base + nfull * 128, rem_rows)],
                    table.at[pl.ds(base + nfull * 128, rem_rows)], gsem1))
                pd.append(pltpu.async_copy(
                    rows_v.at[0, pl.ds(0, rem_rows)],
                    acc.at[pl.ds(base + nfull * 128, rem_rows)], ssem0))
            for d in pd:
                d.wait()
            pltpu.sync_copy(comb_hbm.at[grow0], idx_a)
            plsc.subcore_barrier()

            pltpu.async_copy(table.at[idx_a.at[0]], rows_v.at[0], gsem0)
            lax.fori_loop(0, GQ // 2, pair_body, 0)
            wait_scatter(1)
            wait_scatter(2)
            wait_scatter(3)
            plsc.subcore_barrier()

            wd = [pltpu.async_copy(
                acc.at[pl.ds(base + k * 128, 128)],
                out_hbm.at[cid, h, pl.ds(base + k * 128, 128)], gsem0)
                for k in range(nfull)]
            if rem_rows:
                wd.append(pltpu.async_copy(
                    acc.at[pl.ds(base + nfull * 128, rem_rows)],
                    out_hbm.at[cid, h, pl.ds(base + nfull * 128, rem_rows)],
                    gsem0))
            for d in wd:
                d.wait()
            if h + 1 < npass:
                plsc.subcore_barrier()

    return agg


@functools.lru_cache(None)
def _make_deg():
    F = 64
    mesh = plsc.VectorSubcoreMesh(
        core_axis_name="c", subcore_axis_name="s",
        num_cores=_NC, num_subcores=_NS)
    GQ = _CPT // _KG
    stripe = _NP // _NS

    @functools.partial(
        pl.kernel,
        out_type=jax.ShapeDtypeStruct((_NC, _NP, F), jnp.float32),
        mesh=mesh,
        scratch_types=[
            pltpu.VMEM_SHARED((_NP, F), jnp.float32),
            pltpu.VMEM((_KG, 128), jnp.int32),
            pltpu.VMEM((128, F), jnp.float32),
            pltpu.VMEM((128, F), jnp.float32),
            pltpu.SemaphoreType.DMA,
            pltpu.SemaphoreType.DMA,
        ],
    )
    def deg(didx_hbm, out_hbm, acc, didx_v, ones_v, zero_v, ssem, wsem):
        cid = lax.axis_index("c")
        sid = lax.axis_index("s")
        base = sid * stripe
        nfull = stripe // 128
        rem_rows = stripe - nfull * 128
        zv = jnp.zeros((16,), jnp.float32)
        ov = jnp.ones((16,), jnp.float32)

        def fill_body(r, carry):
            for c in range(F // 16):
                zero_v[r, pl.ds(c * 16, 16)] = zv
                ones_v[r, pl.ds(c * 16, 16)] = ov
            return carry
        lax.fori_loop(0, 128, fill_body, 0)

        zd = [pltpu.async_copy(zero_v,
                               acc.at[pl.ds(base + k * 128, 128)], wsem)
              for k in range(nfull)]
        if rem_rows:
            zd.append(pltpu.async_copy(
                zero_v.at[pl.ds(0, rem_rows)],
                acc.at[pl.ds(base + nfull * 128, rem_rows)], wsem))
        for d in zd:
            d.wait()

        row0 = (cid * _NS + sid) * GQ
        plsc.subcore_barrier()

        def wait_scatter():
            pltpu.make_async_copy(
                ones_v, acc.at[didx_v.at[0]], ssem).wait()

        def group_body(gq, carry):
            pltpu.sync_copy(didx_hbm.at[row0 + gq], didx_v)
            for b in range(_KG):
                pltpu.async_copy(ones_v, acc.at[didx_v.at[b]], ssem,
                                 add=True)
            for b in range(_KG):
                wait_scatter()
            return carry
        lax.fori_loop(0, GQ, group_body, 0)
        plsc.subcore_barrier()

        wd = [pltpu.async_copy(acc.at[pl.ds(base + k * 128, 128)],
                               out_hbm.at[cid, pl.ds(base + k * 128, 128)],
                               wsem)
              for k in range(nfull)]
        if rem_rows:
            wd.append(pltpu.async_copy(
                acc.at[pl.ds(base + nfull * 128, rem_rows)],
                out_hbm.at[cid, pl.ds(base + nfull * 128, rem_rows)], wsem))
        for d in wd:
            d.wait()

    return deg


def _blk3(w):
    return pl.BlockSpec((2, _BLK, w), lambda i: (0, i, 0))


def _blk4():
    return pl.BlockSpec((2, 2, _BLK, 64), lambda i: (0, 0, i, 0))


def _last4():
    return pl.BlockSpec((2, 2, 8, 64), lambda i, _l=_LBI: (0, 0, _l, 0))


def _last3(w):
    lbi = (_N - 1) // 8
    return pl.BlockSpec((2, 8, w), lambda i, _l=lbi: (0, _l, 0))


def _full(shape):
    nd = len(shape)
    return pl.BlockSpec(shape, lambda i, _n=nd: (0,) * _n)


def _fusion_w(a_row, b_row, wa, ba, wb, bb):
    la = jax.nn.sigmoid(jnp.dot(a_row, wa) + ba)
    lb = jax.nn.sigmoid(jnp.dot(b_row, wb) + bb)
    s = jnp.abs(la) + jnp.abs(lb)
    return la / s, lb / s


def _tc_b(x1, x2, W, cnt):
    def body(x1_ref, x2_ref, w_ref, cnt_ref, out_ref):
        dinv = lax.rsqrt(1.0 + cnt_ref[:, :, 0:1])
        w0 = w_ref[0]
        w1 = w_ref[1]
        h1 = jnp.concatenate(
            [jnp.dot(x1_ref[0], w0), jnp.dot(x1_ref[1], w1)], axis=-1)
        h2 = jnp.concatenate(
            [jnp.dot(x2_ref[0], w0), jnp.dot(x2_ref[1], w1)], axis=-1)
        v1 = dinv[0] * h1
        v2 = dinv[1] * h2
        out_ref[0, 0] = v1[:, :64]
        out_ref[0, 1] = v1[:, 64:]
        out_ref[1, 0] = v2[:, :64]
        out_ref[1, 1] = v2[:, 64:]

    return pl.pallas_call(
        body,
        out_shape=jax.ShapeDtypeStruct((_NC, 2, _NP, 64), jnp.float32),
        grid=(_GRID,),
        in_specs=[_blk3(128), _blk3(128), _full((2, 128, 64)), _blk3(64)],
        out_specs=_blk4(),
    )(x1, x2, W, cnt)


def _cat(v):
    return jnp.concatenate([v[:, 0], v[:, 1]], axis=-1)


def _tc_d(S, hs, cnt, bcat, gcat, becat, fw1, fb1, fw2, fb2):
    def body(S_ref, hs_ref, cnt_ref, Sl_ref, hsl_ref, cntl_ref,
             b_ref, g_ref, be_ref, fw1_ref, fb1_ref, fw2_ref, fb2_ref,
             out_ref):
        b = b_ref[...]
        g = g_ref[...]
        be = be_ref[...]

        def xcalc(Sv, hv, cv):
            dinv = lax.rsqrt(1.0 + cv[:, :, 0:1])
            m = dinv * (Sv + hv) + b
            return jax.nn.relu((m * _BNS) * g + be), dinv

        x_blk, dinv = xcalc(_cat(S_ref[...]), _cat(hs_ref[...]), cnt_ref[...])
        x_last, _ = xcalc(_cat(Sl_ref[...]), _cat(hsl_ref[...]), cntl_ref[...])
        w1, w2 = _fusion_w(x_last[0, _LAST:_LAST + 1], x_last[1, _LAST:_LAST + 1],
                           fw1_ref[...], fb1_ref[...], fw2_ref[...], fb2_ref[...])
        xa = w1 * x_blk[0] + w2 * x_blk[1]
        v1 = dinv[0] * xa
        v2 = dinv[1] * xa
        out_ref[0, 0] = v1[:, :64]
        out_ref[0, 1] = v1[:, 64:]
        out_ref[1, 0] = v2[:, :64]
        out_ref[1, 1] = v2[:, 64:]

    return pl.pallas_call(
        body,
        out_shape=jax.ShapeDtypeStruct((_NC, 2, _NP, 64), jnp.float32),
        grid=(_GRID,),
        in_specs=[_blk4(), _blk4(), _blk3(64),
                  _last4(), _last4(), _last3(64),
                  _full((1, 128)), _full((1, 128)), _full((1, 128)),
                  _full((128, 1)), _full((1, 1)), _full((128, 1)), _full((1, 1))],
        out_specs=_blk4(),
    )(S, hs, cnt, S, hs, cnt, bcat, gcat, becat, fw1, fb1, fw2, fb2)


def _tc_f(S, hs2, cnt, Wm, bm, gm, bem, Wo, aw1, ab1, aw2, ab2):
    def body(S_ref, hs_ref, cnt_ref, Sl_ref, hsl_ref, cntl_ref,
             wm_ref, bm_ref, gm_ref, bem_ref, wo_ref,
             aw1_ref, ab1_ref, aw2_ref, ab2_ref, out_ref):
        wm = wm_ref[...]
        bm = bm_ref[...]
        gm = gm_ref[...]
        bem = bem_ref[...]

        def xm(Sv, hv, cv):
            dinv = lax.rsqrt(1.0 + cv[:, :, 0:1])
            m = dinv * (Sv + hv)
            x0 = jax.nn.relu(((jnp.dot(m[0], wm) + bm) * _BNS) * gm + bem)
            x1 = jax.nn.relu(((jnp.dot(m[1], wm) + bm) * _BNS) * gm + bem)
            return x0, x1, dinv

        x0, x1, dinv = xm(_cat(S_ref[...]), _cat(hs_ref[...]), cnt_ref[...])
        xl0, xl1, _ = xm(_cat(Sl_ref[...]), _cat(hsl_ref[...]), cntl_ref[...])
        w1, w2 = _fusion_w(xl0[_LAST:_LAST + 1], xl1[_LAST:_LAST + 1],
                           aw1_ref[...], ab1_ref[...], aw2_ref[...], ab2_ref[...])
        xb = w1 * x0 + w2 * x1
        h = jnp.dot(xb, wo_ref[...])
        v1 = dinv[0] * h
        v2 = dinv[1] * h
        out_ref[0, 0] = v1[:, :64]
        out_ref[0, 1] = v1[:, 64:]
        out_ref[1, 0] = v2[:, :64]
        out_ref[1, 1] = v2[:, 64:]

    return pl.pallas_call(
        body,
        out_shape=jax.ShapeDtypeStruct((_NC, 2, _NP, 64), jnp.float32),
        grid=(_GRID,),
        in_specs=[_blk4(), _blk4(), _blk3(64),
                  _last4(), _last4(), _last3(64),
                  _full((128, 128)), _full((1, 128)), _full((1, 128)),
                  _full((1, 128)), _full((128, 128)),
                  _full((128, 1)), _full((1, 1)), _full((128, 1)), _full((1, 1))],
        out_specs=_blk4(),
    )(S, hs2, cnt, S, hs2, cnt, Wm, bm, gm, bem, Wo, aw1, ab1, aw2, ab2)


def _tc_h(S3, hs3, cnt, bo, fw1, fb1, fw2, fb2):
    def body(S_ref, hs_ref, cnt_ref, Sl_ref, hsl_ref, cntl_ref,
             bo_ref, fw1_ref, fb1_ref, fw2_ref, fb2_ref,
             out_ref, g1_ref, g2_ref):
        bo_v = bo_ref[...]

        def gcalc(Sv, hv, cv):
            dinv = lax.rsqrt(1.0 + cv[:, :, 0:1])
            return (dinv * (Sv[:, 0] + hv[:, 0]))[:, :, :48] + bo_v

        gb = gcalc(S_ref[...], hs_ref[...], cnt_ref[...])
        gl = gcalc(Sl_ref[...], hsl_ref[...], cntl_ref[...])
        w1, w2 = _fusion_w(gl[0, _LAST:_LAST + 1], gl[1, _LAST:_LAST + 1],
                           fw1_ref[...], fb1_ref[...], fw2_ref[...], fb2_ref[...])
        g1_ref[...] = gb[0]
        g2_ref[...] = gb[1]
        out_ref[...] = w1 * gb[0] + w2 * gb[1]

    blk2 = pl.BlockSpec((_BLK, 48), lambda i: (i, 0))
    h1spec = pl.BlockSpec((2, 1, _BLK, 64), lambda i: (0, 0, i, 0))
    h1last = pl.BlockSpec((2, 1, 8, 64), lambda i, _l=_LBI: (0, 0, _l, 0))
    return pl.pallas_call(
        body,
        out_shape=[jax.ShapeDtypeStruct((_NP, 48), jnp.float32)] * 3,
        grid=(_GRID,),
        in_specs=[h1spec, h1spec, _blk3(64),
                  h1last, h1last, _last3(64),
                  _full((1, 48)),
                  _full((48, 1)), _full((1, 1)), _full((48, 1)), _full((1, 1))],
        out_specs=[blk2, blk2, blk2],
    )(S3, hs3, cnt, S3, hs3, cnt, bo, fw1, fb1, fw2, fb2)


def kernel(x_list1, x_list2, adj1, adj2, W_init, b_init, g_init, be_init,
           W_mid, b_mid, g_mid, be_mid, W_out, b_out, fc1w1, fc1b1, fc1w2,
           fc1b2, awsw1, awsb1, awsw2, awsb2, fcw1, fcb1, fcw2, fcb2):
    i32 = jnp.int32

    x1p = jnp.pad(x_list1, ((0, 0), (0, _NP - _N), (0, 0)))
    x2p = jnp.pad(x_list2, ((0, 0), (0, _NP - _N), (0, 0)))

    s1, d1 = adj1[0], adj1[1]
    s2, d2 = adj2[0], adj2[1]
    padv = jnp.full((_EP - _E,), _N, i32)
    gq = _CPT // _KG
    sgr = jnp.concatenate([s1, padv, s2, padv]).reshape(
        _NC * _NS, gq, _KG, 128)
    dgr = jnp.concatenate([d1, padv, d2, padv]).reshape(
        _NC * _NS, gq, _KG, 128)
    comb = jnp.concatenate([sgr, dgr], axis=2).reshape(
        _NC * _NS * gq, 2 * _KG, 128)
    dgc = dgr.reshape(_NC * _NS * gq, _KG, 128)

    agg2p = _make_agg(2)
    agg1p = _make_agg(1)

    cnt = _make_deg()(dgc)

    hsB = _tc_b(x1p, x2p, W_init, cnt)
    S1 = agg2p(hsB, comb)
    hs2 = _tc_d(S1, hsB, cnt,
                b_init.reshape(1, 128), g_init.reshape(1, 128),
                be_init.reshape(1, 128),
                fc1w1, fc1b1.reshape(1, 1), fc1w2, fc1b2.reshape(1, 1))

    S2 = agg2p(hs2, comb)
    Wo_pad = jnp.pad(W_out, ((0, 0), (0, 88)))
    hs3 = _tc_f(S2, hs2, cnt, W_mid,
                b_mid.reshape(1, 128), g_mid.reshape(1, 128),
                be_mid.reshape(1, 128), Wo_pad,
                awsw1, awsb1.reshape(1, 1), awsw2, awsb2.reshape(1, 1))

    S3 = agg1p(hs3, comb)
    bo = jnp.pad(b_out, (0, 8)).reshape(1, 48)
    fw1p = jnp.pad(fcw1, ((0, 8), (0, 0)))
    fw2p = jnp.pad(fcw2, ((0, 8), (0, 0)))
    out, g1, g2 = _tc_h(S3, hs3, cnt, bo,
                        fw1p, fcb1.reshape(1, 1), fw2p, fcb2.reshape(1, 1))
    return (out[:_N, :40], g1[:_N, :40], g2[:_N, :40])
